# Initial kernel scaffold; baseline (speedup 1.0000x reference)
import jax
import jax.numpy as jnp
from jax import lax
from jax.experimental import pallas as pl
from jax.experimental.pallas import tpu as pltpu
from jax.experimental.pallas import tpu_sc as plsc

N = 10000
E = 320000
D = 128

NUM_TILES = 32
EDGES_PER_TILE = E // NUM_TILES
CHUNK = 80
CHUNKS_PER_TILE = EDGES_PER_TILE // CHUNK
ROWS_PER_TILE = N // 16
DEN_PAD = 10240

_ROW_BLOCK = 1000
_GRID = N // _ROW_BLOCK


def _head_body(x_ref, w_ref, as_ref, ad_ref, h_ref, es_ref, ed_ref):
    h = jnp.dot(x_ref[...], w_ref[...], preferred_element_type=jnp.float32)
    h_ref[...] = h
    es_ref[...] = jnp.sum(h * as_ref[...], axis=1)
    ed_ref[...] = jnp.sum(h * ad_ref[...], axis=1)


def _tc_head(x, W, a_s, a_d):
    return pl.pallas_call(
        _head_body,
        grid=(_GRID,),
        in_specs=[
            pl.BlockSpec((_ROW_BLOCK, D), lambda i: (i, 0)),
            pl.BlockSpec((D, D), lambda i: (0, 0)),
            pl.BlockSpec((1, D), lambda i: (0, 0)),
            pl.BlockSpec((1, D), lambda i: (0, 0)),
        ],
        out_specs=[
            pl.BlockSpec((_ROW_BLOCK, D), lambda i: (i, 0)),
            pl.BlockSpec((_ROW_BLOCK,), lambda i: (i,)),
            pl.BlockSpec((_ROW_BLOCK,), lambda i: (i,)),
        ],
        out_shape=[
            jax.ShapeDtypeStruct((N, D), jnp.float32),
            jax.ShapeDtypeStruct((N,), jnp.float32),
            jax.ShapeDtypeStruct((N,), jnp.float32),
        ],
    )(x, W, a_s, a_d)


def _self_p(es, ed):
    e = es + ed
    e = jnp.where(e > 0, e, 0.2 * e)
    return jnp.exp(e)


def _mid_body(na_ref, nb_ref, da_ref, db_ref, h_ref, es_ref, ed_ref, b_ref,
              w_ref, as_ref, ad_ref, h2_ref, es2_ref, ed2_ref):
    h = h_ref[...]
    ps = _self_p(es_ref[...], ed_ref[...])
    num = na_ref[...] + nb_ref[...] + ps[:, None] * h
    den = da_ref[...] + db_ref[...] + ps
    x2 = num / (den[:, None] + 1e-16) + b_ref[...]
    x2 = jnp.maximum(x2, 0.0)
    h2 = jnp.dot(x2, w_ref[...], preferred_element_type=jnp.float32)
    h2_ref[...] = h2
    es2_ref[...] = jnp.sum(h2 * as_ref[...], axis=1)
    ed2_ref[...] = jnp.sum(h2 * ad_ref[...], axis=1)


def _tc_mid(numA, numB, denA, denB, h1, es1, ed1, b1, W2, a_s2, a_d2):
    blk2d = pl.BlockSpec((_ROW_BLOCK, D), lambda i: (i, 0))
    blk1d = pl.BlockSpec((_ROW_BLOCK,), lambda i: (i,))
    full = pl.BlockSpec((D, D), lambda i: (0, 0))
    vec = pl.BlockSpec((1, D), lambda i: (0, 0))
    return pl.pallas_call(
        _mid_body,
        grid=(_GRID,),
        in_specs=[blk2d, blk2d, blk1d, blk1d, blk2d, blk1d, blk1d, vec,
                  full, vec, vec],
        out_specs=[blk2d, blk1d, blk1d],
        out_shape=[
            jax.ShapeDtypeStruct((N, D), jnp.float32),
            jax.ShapeDtypeStruct((N,), jnp.float32),
            jax.ShapeDtypeStruct((N,), jnp.float32),
        ],
    )(numA, numB, denA, denB, h1, es1, ed1, b1, W2, a_s2, a_d2)


def _final_body(na_ref, nb_ref, da_ref, db_ref, h_ref, es_ref, ed_ref, b_ref,
                out_ref):
    h = h_ref[...]
    ps = _self_p(es_ref[...], ed_ref[...])
    num = na_ref[...] + nb_ref[...] + ps[:, None] * h
    den = da_ref[...] + db_ref[...] + ps
    out_ref[...] = num / (den[:, None] + 1e-16) + b_ref[...]


def _tc_final(numA, numB, denA, denB, h2, es2, ed2, b2):
    blk2d = pl.BlockSpec((_ROW_BLOCK, D), lambda i: (i, 0))
    blk1d = pl.BlockSpec((_ROW_BLOCK,), lambda i: (i,))
    vec = pl.BlockSpec((1, D), lambda i: (0, 0))
    return pl.pallas_call(
        _final_body,
        grid=(_GRID,),
        in_specs=[blk2d, blk2d, blk1d, blk1d, blk2d, blk1d, blk1d, vec],
        out_specs=blk2d,
        out_shape=jax.ShapeDtypeStruct((N, D), jnp.float32),
    )(numA, numB, denA, denB, h2, es2, ed2, b2)


def _sc_body(h_hbm, es_hbm, ed_hbm, src_hbm, dst_hbm, src2d_hbm, dst2d_hbm,
             numA_hbm, numB_hbm, denA_hbm, denB_hbm,
             src_v, dst_v, src2d_v, dst2d_v, es_v, ed_v, pch_v, rows_v,
             zden_v, acc_num, acc_den):
    c = lax.axis_index("c")
    s = lax.axis_index("s")
    tile_rows = c * 2000 + s * 125
    edge_base = tile_rows * CHUNK

    pltpu.sync_copy(src_hbm.at[pl.ds(edge_base, EDGES_PER_TILE)], src_v)
    pltpu.sync_copy(dst_hbm.at[pl.ds(edge_base, EDGES_PER_TILE)], dst_v)
    pltpu.sync_copy(src2d_hbm.at[pl.ds(tile_rows, CHUNKS_PER_TILE)], src2d_v)
    pltpu.sync_copy(dst2d_hbm.at[pl.ds(tile_rows, CHUNKS_PER_TILE)], dst2d_v)
    pltpu.sync_copy(es_hbm, es_v)
    pltpu.sync_copy(ed_hbm, ed_v)

    zero16 = jnp.zeros((16,), jnp.float32)
    for r in range(CHUNK):
        for k in range(D // 16):
            rows_v[r, pl.ds(k * 16, 16)] = zero16
    for k in range(640 // 16):
        zden_v[pl.ds(k * 16, 16)] = zero16
    row0 = s * ROWS_PER_TILE
    for k in range(ROWS_PER_TILE // CHUNK):
        pltpu.sync_copy(rows_v, acc_num.at[pl.ds(row0 + k * CHUNK, CHUNK)])
    rem = ROWS_PER_TILE % CHUNK
    pltpu.sync_copy(rows_v.at[pl.ds(0, rem)],
                    acc_num.at[pl.ds(row0 + (ROWS_PER_TILE // CHUNK) * CHUNK,
                                     rem)])
    pltpu.sync_copy(zden_v, acc_den.at[pl.ds(s * 640, 640)])
    plsc.subcore_barrier()

    def chunk_step(j, carry):
        pltpu.sync_copy(h_hbm.at[src2d_v.at[j]], rows_v)
        base = j * CHUNK
        for kk in range(CHUNK // 16):
            s16 = src_v[pl.ds(base + kk * 16, 16)]
            d16 = dst_v[pl.ds(base + kk * 16, 16)]
            e = plsc.load_gather(es_v, [s16]) + plsc.load_gather(ed_v, [d16])
            e = jnp.where(e > 0, e, 0.2 * e)
            pch_v[pl.ds(kk * 16, 16)] = jnp.exp(e)
        for r in range(CHUNK):
            pr = plsc.load_gather(pch_v, [jnp.zeros((16,), jnp.int32) + r])
            for k in range(D // 16):
                rows_v[r, pl.ds(k * 16, 16)] = rows_v[r, pl.ds(k * 16, 16)] * pr
        pltpu.sync_copy(rows_v, acc_num.at[dst2d_v.at[j]], add=True)
        pltpu.sync_copy(pch_v, acc_den.at[dst2d_v.at[j]], add=True)
        return carry

    lax.fori_loop(0, CHUNKS_PER_TILE, chunk_step, 0)
    plsc.subcore_barrier()

    @pl.when(c == 0)
    def _():
        pltpu.sync_copy(acc_num.at[pl.ds(row0, ROWS_PER_TILE)],
                        numA_hbm.at[pl.ds(row0, ROWS_PER_TILE)])
        pltpu.sync_copy(acc_den.at[pl.ds(s * 640, 640)],
                        denA_hbm.at[pl.ds(s * 640, 640)])

    @pl.when(c == 1)
    def _():
        pltpu.sync_copy(acc_num.at[pl.ds(row0, ROWS_PER_TILE)],
                        numB_hbm.at[pl.ds(row0, ROWS_PER_TILE)])
        pltpu.sync_copy(acc_den.at[pl.ds(s * 640, 640)],
                        denB_hbm.at[pl.ds(s * 640, 640)])


_sc_edges = pl.kernel(
    _sc_body,
    out_type=[
        jax.ShapeDtypeStruct((N, D), jnp.float32),
        jax.ShapeDtypeStruct((N, D), jnp.float32),
        jax.ShapeDtypeStruct((DEN_PAD,), jnp.float32),
        jax.ShapeDtypeStruct((DEN_PAD,), jnp.float32),
    ],
    mesh=plsc.VectorSubcoreMesh(core_axis_name="c", subcore_axis_name="s"),
    scratch_types=[
        pltpu.VMEM((EDGES_PER_TILE,), jnp.int32),
        pltpu.VMEM((EDGES_PER_TILE,), jnp.int32),
        pltpu.VMEM((CHUNKS_PER_TILE, CHUNK), jnp.int32),
        pltpu.VMEM((CHUNKS_PER_TILE, CHUNK), jnp.int32),
        pltpu.VMEM((N,), jnp.float32),
        pltpu.VMEM((N,), jnp.float32),
        pltpu.VMEM((CHUNK,), jnp.float32),
        pltpu.VMEM((CHUNK, D), jnp.float32),
        pltpu.VMEM((640,), jnp.float32),
        pltpu.VMEM_SHARED((N, D), jnp.float32),
        pltpu.VMEM_SHARED((DEN_PAD,), jnp.float32),
    ],
)


def kernel(x, edge_index, W1, a_src1, a_dst1, b1, W2, a_src2, a_dst2, b2):
    src = edge_index[0]
    dst = edge_index[1]
    src2d = src.reshape(E // CHUNK, CHUNK)
    dst2d = dst.reshape(E // CHUNK, CHUNK)

    h1, es1, ed1 = _tc_head(x, W1, a_src1.reshape(1, D), a_dst1.reshape(1, D))
    numA, numB, denA, denB = _sc_edges(h1, es1, ed1, src, dst, src2d, dst2d)
    h2, es2, ed2 = _tc_mid(numA, numB, denA[:N], denB[:N], h1, es1, ed1,
                           b1.reshape(1, D), W2, a_src2.reshape(1, D),
                           a_dst2.reshape(1, D))
    numA2, numB2, denA2, denB2 = _sc_edges(h2, es2, ed2, src, dst, src2d, dst2d)
    return _tc_final(numA2, numB2, denA2[:N], denB2[:N], h2, es2, ed2,
                     b2.reshape(1, D))


# speedup vs baseline: 20.4817x; 20.4817x over previous
import functools

import jax
import jax.numpy as jnp
from jax import lax
from jax.experimental import pallas as pl
from jax.experimental.pallas import tpu as pltpu
from jax.experimental.pallas import tpu_sc as plsc

N = 10000
E = 320000
D = 128
DH = D // 2

EDGES_PER_TILE = E // 16
CHUNK = 80
CHUNKS_PER_TILE = EDGES_PER_TILE // CHUNK
N_PAD = 10240
ROWS_PER_TILE = N_PAD // 16

_ROW_BLOCK = 1000
_GRID = N // _ROW_BLOCK


def _head_body(x_ref, w_ref, as_ref, ad_ref, hlo_ref, hhi_ref, es_ref,
               ed_ref):
    h = jnp.dot(x_ref[...], w_ref[...], preferred_element_type=jnp.float32,
                precision=jax.lax.Precision.HIGHEST)
    hlo_ref[...] = h[:, :DH]
    hhi_ref[...] = h[:, DH:]
    es_ref[...] = jnp.sum(h * as_ref[...], axis=1).reshape(1, 1, _ROW_BLOCK)
    ed_ref[...] = jnp.sum(h * ad_ref[...], axis=1).reshape(1, 1, _ROW_BLOCK)


def _tc_head(x, W, a_s, a_d):
    return pl.pallas_call(
        _head_body,
        grid=(_GRID,),
        in_specs=[
            pl.BlockSpec((_ROW_BLOCK, D), lambda i: (i, 0)),
            pl.BlockSpec((D, D), lambda i: (0, 0)),
            pl.BlockSpec((1, D), lambda i: (0, 0)),
            pl.BlockSpec((1, D), lambda i: (0, 0)),
        ],
        out_specs=[
            pl.BlockSpec((_ROW_BLOCK, DH), lambda i: (i, 0)),
            pl.BlockSpec((_ROW_BLOCK, DH), lambda i: (i, 0)),
            pl.BlockSpec((1, 1, _ROW_BLOCK), lambda i: (i, 0, 0)),
            pl.BlockSpec((1, 1, _ROW_BLOCK), lambda i: (i, 0, 0)),
        ],
        out_shape=[
            jax.ShapeDtypeStruct((N, DH), jnp.float32),
            jax.ShapeDtypeStruct((N, DH), jnp.float32),
            jax.ShapeDtypeStruct((_GRID, 1, _ROW_BLOCK), jnp.float32),
            jax.ShapeDtypeStruct((_GRID, 1, _ROW_BLOCK), jnp.float32),
        ],
    )(x, W, a_s, a_d)


def _self_p(es, ed):
    e = es + ed
    e = jnp.where(e > 0, e, 0.2 * e)
    return jnp.exp(e)


def _mid_body(nlo_ref, nhi_ref, dn_ref, hlo_ref, hhi_ref, es_ref, ed_ref,
              b_ref, w_ref, as_ref, ad_ref, h2lo_ref, h2hi_ref, es2_ref,
              ed2_ref):
    h = jnp.concatenate([hlo_ref[...], hhi_ref[...]], axis=1)
    ps = _self_p(es_ref[0, 0, :], ed_ref[0, 0, :])
    num = jnp.concatenate([nlo_ref[...], nhi_ref[...]], axis=1)
    num = num + ps[:, None] * h
    den = dn_ref[0, 0, :] + ps
    x2 = num / (den[:, None] + 1e-16) + b_ref[...]
    x2 = jnp.maximum(x2, 0.0)
    h2 = jnp.dot(x2, w_ref[...], preferred_element_type=jnp.float32,
                 precision=jax.lax.Precision.HIGHEST)
    h2lo_ref[...] = h2[:, :DH]
    h2hi_ref[...] = h2[:, DH:]
    es2_ref[...] = jnp.sum(h2 * as_ref[...], axis=1).reshape(1, 1, _ROW_BLOCK)
    ed2_ref[...] = jnp.sum(h2 * ad_ref[...], axis=1).reshape(1, 1, _ROW_BLOCK)


def _tc_mid(numlo, numhi, den, h1lo, h1hi, es1, ed1, b1, W2, a_s2, a_d2):
    blkh = pl.BlockSpec((_ROW_BLOCK, DH), lambda i: (i, 0))
    blk1d = pl.BlockSpec((1, 1, _ROW_BLOCK), lambda i: (i, 0, 0))
    full = pl.BlockSpec((D, D), lambda i: (0, 0))
    vec = pl.BlockSpec((1, D), lambda i: (0, 0))
    return pl.pallas_call(
        _mid_body,
        grid=(_GRID,),
        in_specs=[blkh, blkh, blk1d, blkh, blkh, blk1d, blk1d, vec,
                  full, vec, vec],
        out_specs=[blkh, blkh, blk1d, blk1d],
        out_shape=[
            jax.ShapeDtypeStruct((N, DH), jnp.float32),
            jax.ShapeDtypeStruct((N, DH), jnp.float32),
            jax.ShapeDtypeStruct((_GRID, 1, _ROW_BLOCK), jnp.float32),
            jax.ShapeDtypeStruct((_GRID, 1, _ROW_BLOCK), jnp.float32),
        ],
    )(numlo, numhi, den, h1lo, h1hi, es1, ed1, b1, W2, a_s2, a_d2)


def _final_body(nlo_ref, nhi_ref, dn_ref, hlo_ref, hhi_ref, es_ref, ed_ref,
                b_ref, out_ref):
    h = jnp.concatenate([hlo_ref[...], hhi_ref[...]], axis=1)
    ps = _self_p(es_ref[0, 0, :], ed_ref[0, 0, :])
    num = jnp.concatenate([nlo_ref[...], nhi_ref[...]], axis=1)
    num = num + ps[:, None] * h
    den = dn_ref[0, 0, :] + ps
    out_ref[...] = num / (den[:, None] + 1e-16) + b_ref[...]


def _tc_final(numlo, numhi, den, h2lo, h2hi, es2, ed2, b2):
    blkh = pl.BlockSpec((_ROW_BLOCK, DH), lambda i: (i, 0))
    blk2d = pl.BlockSpec((_ROW_BLOCK, D), lambda i: (i, 0))
    blk1d = pl.BlockSpec((1, 1, _ROW_BLOCK), lambda i: (i, 0, 0))
    vec = pl.BlockSpec((1, D), lambda i: (0, 0))
    return pl.pallas_call(
        _final_body,
        grid=(_GRID,),
        in_specs=[blkh, blkh, blk1d, blkh, blkh, blk1d, blk1d, vec],
        out_specs=blk2d,
        out_shape=jax.ShapeDtypeStruct((N, D), jnp.float32),
    )(numlo, numhi, den, h2lo, h2hi, es2, ed2, b2)


def _exp_f32(x):
    log2e = jnp.float32(1.4426950408889634)
    y = x * log2e + jnp.float32(12582912.0)
    k = plsc.bitcast(y, jnp.int32) - jnp.int32(0x4B400000)
    kf = (y - jnp.float32(12582912.0))
    r = x - kf * jnp.float32(0.693145751953125)
    r = r - kf * jnp.float32(1.42860677e-06)
    pr = jnp.float32(0.0013888889)
    pr = pr * r + jnp.float32(0.008333333)
    pr = pr * r + jnp.float32(0.041666667)
    pr = pr * r + jnp.float32(0.16666667)
    pr = pr * r + jnp.float32(0.5)
    pr = pr * r + jnp.float32(1.0)
    pr = pr * r + jnp.float32(1.0)
    scale = plsc.bitcast((k + jnp.int32(127)) << jnp.int32(23), jnp.float32)
    return pr * scale


def _sc_body(hlo_hbm, hhi_hbm, es_hbm, ed_hbm, src2d_hbm, dst2d_hbm,
             numlo_hbm, numhi_hbm, den_hbm,
             src2d_v, dst2d_v, es_v, ed_v, pch_v, rows_v, zden_v,
             acc_num, acc_den):
    c = lax.axis_index("c")
    s = lax.axis_index("s")

    pltpu.sync_copy(src2d_hbm.at[s], src2d_v)
    pltpu.sync_copy(dst2d_hbm.at[s], dst2d_v)
    pltpu.sync_copy(es_hbm, es_v)
    pltpu.sync_copy(ed_hbm, ed_v)

    zero16 = jnp.zeros((16,), jnp.float32)
    for r in range(CHUNK):
        for k in range(DH // 16):
            rows_v[r, pl.ds(k * 16, 16)] = zero16
    for k in range(ROWS_PER_TILE // 16):
        zden_v[pl.ds(k * 16, 16)] = zero16
    row0 = s * ROWS_PER_TILE
    for k in range(ROWS_PER_TILE // CHUNK):
        pltpu.sync_copy(rows_v, acc_num.at[pl.ds(row0 + k * CHUNK, CHUNK)])
    pltpu.sync_copy(zden_v, acc_den.at[pl.ds(row0, ROWS_PER_TILE)])
    plsc.subcore_barrier()

    def chunk_step(j, carry):
        @pl.when(c == 0)
        def _():
            pltpu.sync_copy(hlo_hbm.at[src2d_v.at[j]], rows_v)

        @pl.when(c == 1)
        def _():
            pltpu.sync_copy(hhi_hbm.at[src2d_v.at[j]], rows_v)

        for kk in range(CHUNK // 16):
            s16 = src2d_v[j, pl.ds(kk * 16, 16)]
            d16 = dst2d_v[j, pl.ds(kk * 16, 16)]
            e = plsc.load_gather(es_v, [s16]) + plsc.load_gather(ed_v, [d16])
            e = jnp.where(e > 0, e, 0.2 * e)
            pch_v[pl.ds(kk * 16, 16)] = _exp_f32(e)
        for r in range(CHUNK):
            pr = plsc.load_gather(pch_v, [jnp.zeros((16,), jnp.int32) + r])
            for k in range(DH // 16):
                rows_v[r, pl.ds(k * 16, 16)] = rows_v[r, pl.ds(k * 16, 16)] * pr
        pltpu.sync_copy(rows_v, acc_num.at[dst2d_v.at[j]], add=True)

        @pl.when(c == 0)
        def _():
            pltpu.sync_copy(pch_v, acc_den.at[dst2d_v.at[j]], add=True)

        return carry

    lax.fori_loop(0, CHUNKS_PER_TILE, chunk_step, 0)
    plsc.subcore_barrier()

    @pl.when(c == 0)
    def _():
        pltpu.sync_copy(acc_num.at[pl.ds(row0, ROWS_PER_TILE)],
                        numlo_hbm.at[pl.ds(row0, ROWS_PER_TILE)])
        pltpu.sync_copy(acc_den.at[pl.ds(row0, ROWS_PER_TILE)],
                        den_hbm.at[pl.ds(row0, ROWS_PER_TILE)])

    @pl.when(c == 1)
    def _():
        pltpu.sync_copy(acc_num.at[pl.ds(row0, ROWS_PER_TILE)],
                        numhi_hbm.at[pl.ds(row0, ROWS_PER_TILE)])


@functools.lru_cache(maxsize=1)
def _make_sc_edges():
  return pl.kernel(
    _sc_body,
    out_type=[
        jax.ShapeDtypeStruct((N_PAD, DH), jnp.float32),
        jax.ShapeDtypeStruct((N_PAD, DH), jnp.float32),
        jax.ShapeDtypeStruct((N_PAD,), jnp.float32),
    ],
    mesh=plsc.VectorSubcoreMesh(core_axis_name="c", subcore_axis_name="s"),
    compiler_params=pltpu.CompilerParams(needs_layout_passes=False,
                                         use_tc_tiling_on_sc=False),
    scratch_types=[
        pltpu.VMEM((CHUNKS_PER_TILE, CHUNK), jnp.int32),
        pltpu.VMEM((CHUNKS_PER_TILE, CHUNK), jnp.int32),
        pltpu.VMEM((N,), jnp.float32),
        pltpu.VMEM((N,), jnp.float32),
        pltpu.VMEM((CHUNK,), jnp.float32),
        pltpu.VMEM((CHUNK, DH), jnp.float32),
        pltpu.VMEM((ROWS_PER_TILE,), jnp.float32),
        pltpu.VMEM_SHARED((N_PAD, DH), jnp.float32),
        pltpu.VMEM_SHARED((N_PAD,), jnp.float32),
    ],
  )


def kernel(x, edge_index, W1, a_src1, a_dst1, b1, W2, a_src2, a_dst2, b2):
    src = edge_index[0]
    dst = edge_index[1]
    src2d = src.reshape(16, CHUNKS_PER_TILE, CHUNK)
    dst2d = dst.reshape(16, CHUNKS_PER_TILE, CHUNK)
    g3 = (_GRID, 1, _ROW_BLOCK)

    h1lo, h1hi, es1, ed1 = _tc_head(x, W1, a_src1.reshape(1, D),
                                    a_dst1.reshape(1, D))
    nlo, nhi, den = _make_sc_edges()(h1lo, h1hi, es1.reshape(N),
                                     ed1.reshape(N), src2d, dst2d)
    h2lo, h2hi, es2, ed2 = _tc_mid(nlo[:N], nhi[:N], den[:N].reshape(g3),
                                   h1lo, h1hi, es1, ed1, b1.reshape(1, D),
                                   W2, a_src2.reshape(1, D),
                                   a_dst2.reshape(1, D))
    nlo2, nhi2, den2 = _make_sc_edges()(h2lo, h2hi, es2.reshape(N),
                                        ed2.reshape(N), src2d, dst2d)
    return _tc_final(nlo2[:N], nhi2[:N], den2[:N].reshape(g3), h2lo, h2hi,
                     es2, ed2, b2.reshape(1, D))


# speedup vs baseline: 24.1393x; 1.1786x over previous
import functools

import jax
import jax.numpy as jnp
from jax import lax
from jax.experimental import pallas as pl
from jax.experimental.pallas import tpu as pltpu
from jax.experimental.pallas import tpu_sc as plsc

N = 10000
E = 320000
D = 128
DH = D // 2

EDGES_PER_TILE = E // 16
CHUNK = 80
CHUNKS_PER_TILE = EDGES_PER_TILE // CHUNK
N_PAD = 10240
ROWS_PER_TILE = N_PAD // 16

_ROW_BLOCK = 1000
_GRID = N // _ROW_BLOCK


def _head_body(x_ref, w_ref, as_ref, ad_ref, hlo_ref, hhi_ref, es_ref,
               ed_ref):
    h = jnp.dot(x_ref[...], w_ref[...], preferred_element_type=jnp.float32,
                precision=jax.lax.Precision.HIGHEST)
    hlo_ref[...] = h[:, :DH]
    hhi_ref[...] = h[:, DH:]
    es_ref[...] = jnp.sum(h * as_ref[...], axis=1).reshape(1, 1, _ROW_BLOCK)
    ed_ref[...] = jnp.sum(h * ad_ref[...], axis=1).reshape(1, 1, _ROW_BLOCK)


def _tc_head(x, W, a_s, a_d):
    return pl.pallas_call(
        _head_body,
        grid=(_GRID,),
        in_specs=[
            pl.BlockSpec((_ROW_BLOCK, D), lambda i: (i, 0)),
            pl.BlockSpec((D, D), lambda i: (0, 0)),
            pl.BlockSpec((1, D), lambda i: (0, 0)),
            pl.BlockSpec((1, D), lambda i: (0, 0)),
        ],
        out_specs=[
            pl.BlockSpec((_ROW_BLOCK, DH), lambda i: (i, 0)),
            pl.BlockSpec((_ROW_BLOCK, DH), lambda i: (i, 0)),
            pl.BlockSpec((1, 1, _ROW_BLOCK), lambda i: (i, 0, 0)),
            pl.BlockSpec((1, 1, _ROW_BLOCK), lambda i: (i, 0, 0)),
        ],
        out_shape=[
            jax.ShapeDtypeStruct((N, DH), jnp.float32),
            jax.ShapeDtypeStruct((N, DH), jnp.float32),
            jax.ShapeDtypeStruct((_GRID, 1, _ROW_BLOCK), jnp.float32),
            jax.ShapeDtypeStruct((_GRID, 1, _ROW_BLOCK), jnp.float32),
        ],
    )(x, W, a_s, a_d)


def _self_p(es, ed):
    e = es + ed
    e = jnp.where(e > 0, e, 0.2 * e)
    return jnp.exp(e)


def _mid_body(nlo_ref, nhi_ref, dn_ref, hlo_ref, hhi_ref, es_ref, ed_ref,
              b_ref, w_ref, as_ref, ad_ref, h2lo_ref, h2hi_ref, es2_ref,
              ed2_ref):
    h = jnp.concatenate([hlo_ref[...], hhi_ref[...]], axis=1)
    ps = _self_p(es_ref[0, 0, :], ed_ref[0, 0, :])
    num = jnp.concatenate([nlo_ref[...], nhi_ref[...]], axis=1)
    num = num + ps[:, None] * h
    den = dn_ref[0, 0, :] + ps
    x2 = num / (den[:, None] + 1e-16) + b_ref[...]
    x2 = jnp.maximum(x2, 0.0)
    h2 = jnp.dot(x2, w_ref[...], preferred_element_type=jnp.float32,
                 precision=jax.lax.Precision.HIGHEST)
    h2lo_ref[...] = h2[:, :DH]
    h2hi_ref[...] = h2[:, DH:]
    es2_ref[...] = jnp.sum(h2 * as_ref[...], axis=1).reshape(1, 1, _ROW_BLOCK)
    ed2_ref[...] = jnp.sum(h2 * ad_ref[...], axis=1).reshape(1, 1, _ROW_BLOCK)


def _tc_mid(numlo, numhi, den, h1lo, h1hi, es1, ed1, b1, W2, a_s2, a_d2):
    blkh = pl.BlockSpec((_ROW_BLOCK, DH), lambda i: (i, 0))
    blk1d = pl.BlockSpec((1, 1, _ROW_BLOCK), lambda i: (i, 0, 0))
    full = pl.BlockSpec((D, D), lambda i: (0, 0))
    vec = pl.BlockSpec((1, D), lambda i: (0, 0))
    return pl.pallas_call(
        _mid_body,
        grid=(_GRID,),
        in_specs=[blkh, blkh, blk1d, blkh, blkh, blk1d, blk1d, vec,
                  full, vec, vec],
        out_specs=[blkh, blkh, blk1d, blk1d],
        out_shape=[
            jax.ShapeDtypeStruct((N, DH), jnp.float32),
            jax.ShapeDtypeStruct((N, DH), jnp.float32),
            jax.ShapeDtypeStruct((_GRID, 1, _ROW_BLOCK), jnp.float32),
            jax.ShapeDtypeStruct((_GRID, 1, _ROW_BLOCK), jnp.float32),
        ],
    )(numlo, numhi, den, h1lo, h1hi, es1, ed1, b1, W2, a_s2, a_d2)


def _final_body(nlo_ref, nhi_ref, dn_ref, hlo_ref, hhi_ref, es_ref, ed_ref,
                b_ref, out_ref):
    h = jnp.concatenate([hlo_ref[...], hhi_ref[...]], axis=1)
    ps = _self_p(es_ref[0, 0, :], ed_ref[0, 0, :])
    num = jnp.concatenate([nlo_ref[...], nhi_ref[...]], axis=1)
    num = num + ps[:, None] * h
    den = dn_ref[0, 0, :] + ps
    out_ref[...] = num / (den[:, None] + 1e-16) + b_ref[...]


def _tc_final(numlo, numhi, den, h2lo, h2hi, es2, ed2, b2):
    blkh = pl.BlockSpec((_ROW_BLOCK, DH), lambda i: (i, 0))
    blk2d = pl.BlockSpec((_ROW_BLOCK, D), lambda i: (i, 0))
    blk1d = pl.BlockSpec((1, 1, _ROW_BLOCK), lambda i: (i, 0, 0))
    vec = pl.BlockSpec((1, D), lambda i: (0, 0))
    return pl.pallas_call(
        _final_body,
        grid=(_GRID,),
        in_specs=[blkh, blkh, blk1d, blkh, blkh, blk1d, blk1d, vec],
        out_specs=blk2d,
        out_shape=jax.ShapeDtypeStruct((N, D), jnp.float32),
    )(numlo, numhi, den, h2lo, h2hi, es2, ed2, b2)


def _exp_f32(x):
    log2e = jnp.float32(1.4426950408889634)
    y = x * log2e + jnp.float32(12582912.0)
    k = plsc.bitcast(y, jnp.int32) - jnp.int32(0x4B400000)
    kf = (y - jnp.float32(12582912.0))
    r = x - kf * jnp.float32(0.693145751953125)
    r = r - kf * jnp.float32(1.42860677e-06)
    pr = jnp.float32(0.0013888889)
    pr = pr * r + jnp.float32(0.008333333)
    pr = pr * r + jnp.float32(0.041666667)
    pr = pr * r + jnp.float32(0.16666667)
    pr = pr * r + jnp.float32(0.5)
    pr = pr * r + jnp.float32(1.0)
    pr = pr * r + jnp.float32(1.0)
    scale = plsc.bitcast((k + jnp.int32(127)) << jnp.int32(23), jnp.float32)
    return pr * scale


def _sc_body(hlo_hbm, hhi_hbm, es_hbm, ed_hbm, src2d_hbm, dst2d_hbm,
             numlo_hbm, numhi_hbm, den_hbm,
             src2d_v, dst2d_v, es_v, ed_v, pch_v, rows_v, zden_v,
             acc_num, acc_den):
    c = lax.axis_index("c")
    s = lax.axis_index("s")

    pltpu.sync_copy(src2d_hbm.at[s], src2d_v)
    pltpu.sync_copy(dst2d_hbm.at[s], dst2d_v)
    pltpu.sync_copy(es_hbm, es_v)
    pltpu.sync_copy(ed_hbm, ed_v)

    zero16 = jnp.zeros((16,), jnp.float32)
    for r in range(CHUNK):
        for k in range(DH // 16):
            rows_v[r, pl.ds(k * 16, 16)] = zero16
    for k in range(ROWS_PER_TILE // 16):
        zden_v[pl.ds(k * 16, 16)] = zero16
    row0 = s * ROWS_PER_TILE
    for k in range(ROWS_PER_TILE // CHUNK):
        pltpu.sync_copy(rows_v, acc_num.at[pl.ds(row0 + k * CHUNK, CHUNK)])
    pltpu.sync_copy(zden_v, acc_den.at[pl.ds(row0, ROWS_PER_TILE)])
    plsc.subcore_barrier()

    def chunk_step(j, carry):
        @pl.when(c == 0)
        def _():
            pltpu.sync_copy(hlo_hbm.at[src2d_v.at[j]], rows_v)

        @pl.when(c == 1)
        def _():
            pltpu.sync_copy(hhi_hbm.at[src2d_v.at[j]], rows_v)

        for kk in range(CHUNK // 16):
            s16 = src2d_v[j, pl.ds(kk * 16, 16)]
            d16 = dst2d_v[j, pl.ds(kk * 16, 16)]
            e = plsc.load_gather(es_v, [s16]) + plsc.load_gather(ed_v, [d16])
            e = jnp.where(e > 0, e, 0.2 * e)
            p16 = _exp_f32(e)
            pch_v[pl.ds(kk * 16, 16)] = p16
            for i in range(16):
                r = kk * 16 + i
                pr = jnp.broadcast_to(p16[i], (16,))
                for k in range(DH // 16):
                    rows_v[r, pl.ds(k * 16, 16)] = (
                        rows_v[r, pl.ds(k * 16, 16)] * pr)
        pltpu.sync_copy(rows_v, acc_num.at[dst2d_v.at[j]], add=True)

        @pl.when(c == 0)
        def _():
            pltpu.sync_copy(pch_v, acc_den.at[dst2d_v.at[j]], add=True)

        return carry

    lax.fori_loop(0, CHUNKS_PER_TILE, chunk_step, 0)
    plsc.subcore_barrier()

    @pl.when(c == 0)
    def _():
        pltpu.sync_copy(acc_num.at[pl.ds(row0, ROWS_PER_TILE)],
                        numlo_hbm.at[pl.ds(row0, ROWS_PER_TILE)])
        pltpu.sync_copy(acc_den.at[pl.ds(row0, ROWS_PER_TILE)],
                        den_hbm.at[pl.ds(row0, ROWS_PER_TILE)])

    @pl.when(c == 1)
    def _():
        pltpu.sync_copy(acc_num.at[pl.ds(row0, ROWS_PER_TILE)],
                        numhi_hbm.at[pl.ds(row0, ROWS_PER_TILE)])


@functools.lru_cache(maxsize=1)
def _make_sc_edges():
  return pl.kernel(
    _sc_body,
    out_type=[
        jax.ShapeDtypeStruct((N_PAD, DH), jnp.float32),
        jax.ShapeDtypeStruct((N_PAD, DH), jnp.float32),
        jax.ShapeDtypeStruct((N_PAD,), jnp.float32),
    ],
    mesh=plsc.VectorSubcoreMesh(core_axis_name="c", subcore_axis_name="s"),
    compiler_params=pltpu.CompilerParams(needs_layout_passes=False,
                                         use_tc_tiling_on_sc=False),
    scratch_types=[
        pltpu.VMEM((CHUNKS_PER_TILE, CHUNK), jnp.int32),
        pltpu.VMEM((CHUNKS_PER_TILE, CHUNK), jnp.int32),
        pltpu.VMEM((N,), jnp.float32),
        pltpu.VMEM((N,), jnp.float32),
        pltpu.VMEM((CHUNK,), jnp.float32),
        pltpu.VMEM((CHUNK, DH), jnp.float32),
        pltpu.VMEM((ROWS_PER_TILE,), jnp.float32),
        pltpu.VMEM_SHARED((N_PAD, DH), jnp.float32),
        pltpu.VMEM_SHARED((N_PAD,), jnp.float32),
    ],
  )


def kernel(x, edge_index, W1, a_src1, a_dst1, b1, W2, a_src2, a_dst2, b2):
    src = edge_index[0]
    dst = edge_index[1]
    src2d = src.reshape(16, CHUNKS_PER_TILE, CHUNK)
    dst2d = dst.reshape(16, CHUNKS_PER_TILE, CHUNK)
    g3 = (_GRID, 1, _ROW_BLOCK)

    h1lo, h1hi, es1, ed1 = _tc_head(x, W1, a_src1.reshape(1, D),
                                    a_dst1.reshape(1, D))
    nlo, nhi, den = _make_sc_edges()(h1lo, h1hi, es1.reshape(N),
                                     ed1.reshape(N), src2d, dst2d)
    h2lo, h2hi, es2, ed2 = _tc_mid(nlo[:N], nhi[:N], den[:N].reshape(g3),
                                   h1lo, h1hi, es1, ed1, b1.reshape(1, D),
                                   W2, a_src2.reshape(1, D),
                                   a_dst2.reshape(1, D))
    nlo2, nhi2, den2 = _make_sc_edges()(h2lo, h2hi, es2.reshape(N),
                                        ed2.reshape(N), src2d, dst2d)
    return _tc_final(nlo2[:N], nhi2[:N], den2[:N].reshape(g3), h2lo, h2hi,
                     es2, ed2, b2.reshape(1, D))


# speedup vs baseline: 39.0194x; 1.6164x over previous
import functools

import jax
import jax.numpy as jnp
from jax import lax
from jax.experimental import pallas as pl
from jax.experimental.pallas import tpu as pltpu
from jax.experimental.pallas import tpu_sc as plsc

N = 10000
E = 320000
D = 128
DH = D // 2

EDGES_PER_TILE = E // 16
CHUNK = 80
CHUNKS_PER_TILE = EDGES_PER_TILE // CHUNK
N_PAD = 10240
ROWS_PER_TILE = N_PAD // 16

_ROW_BLOCK = 1000
_GRID = N // _ROW_BLOCK


def _head_body(x_ref, w_ref, as_ref, ad_ref, hlo_ref, hhi_ref, es_ref,
               ed_ref):
    h = jnp.dot(x_ref[...], w_ref[...], preferred_element_type=jnp.float32,
                precision=jax.lax.Precision.HIGHEST)
    hlo_ref[...] = h[:, :DH]
    hhi_ref[...] = h[:, DH:]
    es_ref[...] = jnp.sum(h * as_ref[...], axis=1).reshape(1, 1, _ROW_BLOCK)
    ed_ref[...] = jnp.sum(h * ad_ref[...], axis=1).reshape(1, 1, _ROW_BLOCK)


def _tc_head(x, W, a_s, a_d):
    return pl.pallas_call(
        _head_body,
        grid=(_GRID,),
        in_specs=[
            pl.BlockSpec((_ROW_BLOCK, D), lambda i: (i, 0)),
            pl.BlockSpec((D, D), lambda i: (0, 0)),
            pl.BlockSpec((1, D), lambda i: (0, 0)),
            pl.BlockSpec((1, D), lambda i: (0, 0)),
        ],
        out_specs=[
            pl.BlockSpec((_ROW_BLOCK, DH), lambda i: (i, 0)),
            pl.BlockSpec((_ROW_BLOCK, DH), lambda i: (i, 0)),
            pl.BlockSpec((1, 1, _ROW_BLOCK), lambda i: (i, 0, 0)),
            pl.BlockSpec((1, 1, _ROW_BLOCK), lambda i: (i, 0, 0)),
        ],
        out_shape=[
            jax.ShapeDtypeStruct((N, DH), jnp.float32),
            jax.ShapeDtypeStruct((N, DH), jnp.float32),
            jax.ShapeDtypeStruct((_GRID, 1, _ROW_BLOCK), jnp.float32),
            jax.ShapeDtypeStruct((_GRID, 1, _ROW_BLOCK), jnp.float32),
        ],
    )(x, W, a_s, a_d)


def _self_p(es, ed):
    e = es + ed
    e = jnp.where(e > 0, e, 0.2 * e)
    return jnp.exp(e)


def _mid_body(nlo_ref, nhi_ref, dn_ref, dn1_ref, hlo_ref, hhi_ref, es_ref,
              ed_ref, b_ref, w_ref, as_ref, ad_ref, h2lo_ref, h2hi_ref,
              es2_ref, ed2_ref):
    h = jnp.concatenate([hlo_ref[...], hhi_ref[...]], axis=1)
    ps = _self_p(es_ref[0, 0, :], ed_ref[0, 0, :])
    num = jnp.concatenate([nlo_ref[...], nhi_ref[...]], axis=1)
    num = num + ps[:, None] * h
    den = dn_ref[0, 0, :] + dn1_ref[0, 0, :] + ps
    x2 = num / (den[:, None] + 1e-16) + b_ref[...]
    x2 = jnp.maximum(x2, 0.0)
    h2 = jnp.dot(x2, w_ref[...], preferred_element_type=jnp.float32,
                 precision=jax.lax.Precision.HIGHEST)
    h2lo_ref[...] = h2[:, :DH]
    h2hi_ref[...] = h2[:, DH:]
    es2_ref[...] = jnp.sum(h2 * as_ref[...], axis=1).reshape(1, 1, _ROW_BLOCK)
    ed2_ref[...] = jnp.sum(h2 * ad_ref[...], axis=1).reshape(1, 1, _ROW_BLOCK)


def _tc_mid(numlo, numhi, den, den1, h1lo, h1hi, es1, ed1, b1, W2, a_s2,
            a_d2):
    blkh = pl.BlockSpec((_ROW_BLOCK, DH), lambda i: (i, 0))
    blk1d = pl.BlockSpec((1, 1, _ROW_BLOCK), lambda i: (i, 0, 0))
    full = pl.BlockSpec((D, D), lambda i: (0, 0))
    vec = pl.BlockSpec((1, D), lambda i: (0, 0))
    return pl.pallas_call(
        _mid_body,
        grid=(_GRID,),
        in_specs=[blkh, blkh, blk1d, blk1d, blkh, blkh, blk1d, blk1d, vec,
                  full, vec, vec],
        out_specs=[blkh, blkh, blk1d, blk1d],
        out_shape=[
            jax.ShapeDtypeStruct((N, DH), jnp.float32),
            jax.ShapeDtypeStruct((N, DH), jnp.float32),
            jax.ShapeDtypeStruct((_GRID, 1, _ROW_BLOCK), jnp.float32),
            jax.ShapeDtypeStruct((_GRID, 1, _ROW_BLOCK), jnp.float32),
        ],
    )(numlo, numhi, den, den1, h1lo, h1hi, es1, ed1, b1, W2, a_s2, a_d2)


def _final_body(nlo_ref, nhi_ref, dn_ref, dn1_ref, hlo_ref, hhi_ref, es_ref,
                ed_ref, b_ref, out_ref):
    h = jnp.concatenate([hlo_ref[...], hhi_ref[...]], axis=1)
    ps = _self_p(es_ref[0, 0, :], ed_ref[0, 0, :])
    num = jnp.concatenate([nlo_ref[...], nhi_ref[...]], axis=1)
    num = num + ps[:, None] * h
    den = dn_ref[0, 0, :] + dn1_ref[0, 0, :] + ps
    out_ref[...] = num / (den[:, None] + 1e-16) + b_ref[...]


def _tc_final(numlo, numhi, den, den1, h2lo, h2hi, es2, ed2, b2):
    blkh = pl.BlockSpec((_ROW_BLOCK, DH), lambda i: (i, 0))
    blk2d = pl.BlockSpec((_ROW_BLOCK, D), lambda i: (i, 0))
    blk1d = pl.BlockSpec((1, 1, _ROW_BLOCK), lambda i: (i, 0, 0))
    vec = pl.BlockSpec((1, D), lambda i: (0, 0))
    return pl.pallas_call(
        _final_body,
        grid=(_GRID,),
        in_specs=[blkh, blkh, blk1d, blk1d, blkh, blkh, blk1d, blk1d, vec],
        out_specs=blk2d,
        out_shape=jax.ShapeDtypeStruct((N, D), jnp.float32),
    )(numlo, numhi, den, den1, h2lo, h2hi, es2, ed2, b2)


def _exp_f32(x):
    log2e = jnp.float32(1.4426950408889634)
    y = x * log2e + jnp.float32(12582912.0)
    k = plsc.bitcast(y, jnp.int32) - jnp.int32(0x4B400000)
    kf = (y - jnp.float32(12582912.0))
    r = x - kf * jnp.float32(0.693145751953125)
    r = r - kf * jnp.float32(1.42860677e-06)
    pr = jnp.float32(0.0013888889)
    pr = pr * r + jnp.float32(0.008333333)
    pr = pr * r + jnp.float32(0.041666667)
    pr = pr * r + jnp.float32(0.16666667)
    pr = pr * r + jnp.float32(0.5)
    pr = pr * r + jnp.float32(1.0)
    pr = pr * r + jnp.float32(1.0)
    scale = plsc.bitcast((k + jnp.int32(127)) << jnp.int32(23), jnp.float32)
    return pr * scale


def _sc_body(hlo_hbm, hhi_hbm, es_hbm, ed_hbm, src2d_hbm, dst2d_hbm,
             numlo_hbm, numhi_hbm, den0_hbm, den1_hbm,
             src2d_v, dst2d_v, es_v, ed_v, pch_v, rows_v, zden_v,
             acc_num, acc_den, gsem0, gsem1, ssem0, ssem1, dsem0, dsem1):
    c = lax.axis_index("c")
    s = lax.axis_index("s")

    pltpu.sync_copy(src2d_hbm.at[s], src2d_v)
    pltpu.sync_copy(dst2d_hbm.at[s], dst2d_v)
    pltpu.sync_copy(es_hbm, es_v)
    pltpu.sync_copy(ed_hbm, ed_v)

    zero16 = jnp.zeros((16,), jnp.float32)
    for r in range(CHUNK):
        for k in range(DH // 16):
            rows_v[0, r, pl.ds(k * 16, 16)] = zero16
    for k in range(ROWS_PER_TILE // 16):
        zden_v[pl.ds(k * 16, 16)] = zero16
    row0 = s * ROWS_PER_TILE
    for k in range(ROWS_PER_TILE // CHUNK):
        pltpu.sync_copy(rows_v.at[0],
                        acc_num.at[pl.ds(row0 + k * CHUNK, CHUNK)])
    pltpu.sync_copy(zden_v, acc_den.at[pl.ds(row0, ROWS_PER_TILE)])
    plsc.subcore_barrier()

    def _compute_scale(b, j):
        for kk in range(CHUNK // 16):
            s16 = src2d_v[j, pl.ds(kk * 16, 16)]
            d16 = dst2d_v[j, pl.ds(kk * 16, 16)]
            e = plsc.load_gather(es_v, [s16]) + plsc.load_gather(ed_v, [d16])
            e = jnp.where(e > 0, e, 0.2 * e)
            p16 = _exp_f32(e)
            pch_v[b, pl.ds(kk * 16, 16)] = p16
            for i in range(16):
                r = kk * 16 + i
                pr = jnp.broadcast_to(p16[i], (16,))
                for k in range(DH // 16):
                    rows_v[b, r, pl.ds(k * 16, 16)] = (
                        rows_v[b, r, pl.ds(k * 16, 16)] * pr)

    def _fire_gather(b, j, sem):
        @pl.when(c == 0)
        def _():
            pltpu.async_copy(hlo_hbm.at[src2d_v.at[j]], rows_v.at[b], sem)

        @pl.when(c == 1)
        def _():
            pltpu.async_copy(hhi_hbm.at[src2d_v.at[j]], rows_v.at[b], sem)

    def _wait_gather(b, j, sem):
        pltpu.make_async_copy(hlo_hbm.at[src2d_v.at[j]], rows_v.at[b],
                              sem).wait()

    _fire_gather(0, 0, gsem0)
    _fire_gather(1, 1, gsem1)

    def pair_step(q, carry):
        j0 = 2 * q
        j1 = 2 * q + 1
        _wait_gather(0, j0, gsem0)
        _compute_scale(0, j0)
        s0 = pltpu.async_copy(rows_v.at[0], acc_num.at[dst2d_v.at[j0]],
                              ssem0, add=True)

        @pl.when(c == 0)
        def _():
            pltpu.async_copy(pch_v.at[0], acc_den.at[dst2d_v.at[j0]],
                             dsem0, add=True)

        _wait_gather(1, j1, gsem1)
        _compute_scale(1, j1)
        s1 = pltpu.async_copy(rows_v.at[1], acc_num.at[dst2d_v.at[j1]],
                              ssem1, add=True)

        @pl.when(c == 1)
        def _():
            pltpu.async_copy(pch_v.at[1], acc_den.at[dst2d_v.at[j1]],
                             dsem1, add=True)

        s0.wait()

        @pl.when(q < CHUNKS_PER_TILE // 2 - 1)
        def _():
            _fire_gather(0, j0 + 2, gsem0)

        s1.wait()

        @pl.when(q < CHUNKS_PER_TILE // 2 - 1)
        def _():
            _fire_gather(1, j1 + 2, gsem1)

        @pl.when(c == 0)
        def _():
            pltpu.make_async_copy(pch_v.at[0], acc_den.at[dst2d_v.at[j0]],
                                  dsem0).wait()

        @pl.when(c == 1)
        def _():
            pltpu.make_async_copy(pch_v.at[1], acc_den.at[dst2d_v.at[j1]],
                                  dsem1).wait()

        return carry

    lax.fori_loop(0, CHUNKS_PER_TILE // 2, pair_step, 0)
    plsc.subcore_barrier()

    @pl.when(c == 0)
    def _():
        @pl.when(s < 15)
        def _():
            pltpu.sync_copy(acc_num.at[pl.ds(row0, ROWS_PER_TILE)],
                            numlo_hbm.at[pl.ds(row0, ROWS_PER_TILE)])
            pltpu.sync_copy(acc_den.at[pl.ds(row0, ROWS_PER_TILE)],
                            den0_hbm.at[pl.ds(row0, ROWS_PER_TILE)])

        @pl.when(s == 15)
        def _():
            pltpu.sync_copy(acc_num.at[pl.ds(row0, N - 15 * ROWS_PER_TILE)],
                            numlo_hbm.at[pl.ds(row0,
                                               N - 15 * ROWS_PER_TILE)])
            pltpu.sync_copy(acc_den.at[pl.ds(row0, N - 15 * ROWS_PER_TILE)],
                            den0_hbm.at[pl.ds(row0, N - 15 * ROWS_PER_TILE)])

    @pl.when(c == 1)
    def _():
        @pl.when(s < 15)
        def _():
            pltpu.sync_copy(acc_num.at[pl.ds(row0, ROWS_PER_TILE)],
                            numhi_hbm.at[pl.ds(row0, ROWS_PER_TILE)])
            pltpu.sync_copy(acc_den.at[pl.ds(row0, ROWS_PER_TILE)],
                            den1_hbm.at[pl.ds(row0, ROWS_PER_TILE)])

        @pl.when(s == 15)
        def _():
            pltpu.sync_copy(acc_num.at[pl.ds(row0, N - 15 * ROWS_PER_TILE)],
                            numhi_hbm.at[pl.ds(row0,
                                               N - 15 * ROWS_PER_TILE)])
            pltpu.sync_copy(acc_den.at[pl.ds(row0, N - 15 * ROWS_PER_TILE)],
                            den1_hbm.at[pl.ds(row0, N - 15 * ROWS_PER_TILE)])


@functools.lru_cache(maxsize=1)
def _make_sc_edges():
  return pl.kernel(
    _sc_body,
    out_type=[
        jax.ShapeDtypeStruct((N, DH), jnp.float32),
        jax.ShapeDtypeStruct((N, DH), jnp.float32),
        jax.ShapeDtypeStruct((N,), jnp.float32),
        jax.ShapeDtypeStruct((N,), jnp.float32),
    ],
    mesh=plsc.VectorSubcoreMesh(core_axis_name="c", subcore_axis_name="s"),
    compiler_params=pltpu.CompilerParams(needs_layout_passes=False,
                                         use_tc_tiling_on_sc=False),
    scratch_types=[
        pltpu.VMEM((CHUNKS_PER_TILE, CHUNK), jnp.int32),
        pltpu.VMEM((CHUNKS_PER_TILE, CHUNK), jnp.int32),
        pltpu.VMEM((N,), jnp.float32),
        pltpu.VMEM((N,), jnp.float32),
        pltpu.VMEM((2, CHUNK), jnp.float32),
        pltpu.VMEM((2, CHUNK, DH), jnp.float32),
        pltpu.VMEM((ROWS_PER_TILE,), jnp.float32),
        pltpu.VMEM_SHARED((N_PAD, DH), jnp.float32),
        pltpu.VMEM_SHARED((N_PAD,), jnp.float32),
        pltpu.SemaphoreType.DMA,
        pltpu.SemaphoreType.DMA,
        pltpu.SemaphoreType.DMA,
        pltpu.SemaphoreType.DMA,
        pltpu.SemaphoreType.DMA,
        pltpu.SemaphoreType.DMA,
    ],
  )


def kernel(x, edge_index, W1, a_src1, a_dst1, b1, W2, a_src2, a_dst2, b2):
    src = edge_index[0]
    dst = edge_index[1]
    src2d = src.reshape(16, CHUNKS_PER_TILE, CHUNK)
    dst2d = dst.reshape(16, CHUNKS_PER_TILE, CHUNK)
    g3 = (_GRID, 1, _ROW_BLOCK)

    h1lo, h1hi, es1, ed1 = _tc_head(x, W1, a_src1.reshape(1, D),
                                    a_dst1.reshape(1, D))
    nlo, nhi, den, denb = _make_sc_edges()(h1lo, h1hi, es1.reshape(N),
                                           ed1.reshape(N), src2d, dst2d)
    h2lo, h2hi, es2, ed2 = _tc_mid(nlo, nhi, den.reshape(g3),
                                   denb.reshape(g3), h1lo, h1hi, es1,
                                   ed1, b1.reshape(1, D), W2,
                                   a_src2.reshape(1, D),
                                   a_dst2.reshape(1, D))
    nlo2, nhi2, den2, den2b = _make_sc_edges()(h2lo, h2hi, es2.reshape(N),
                                               ed2.reshape(N), src2d, dst2d)
    return _tc_final(nlo2, nhi2, den2.reshape(g3),
                     den2b.reshape(g3), h2lo, h2hi, es2, ed2,
                     b2.reshape(1, D))
